# Initial kernel scaffold; baseline (speedup 1.0000x reference)
import jax
import jax.numpy as jnp
from jax.experimental import pallas as pl


def kernel(x, segment_info, table, seg_table, pos):
    raise NotImplementedError("write your pallas kernel here")


# speedup vs baseline: 1.1507x; 1.1507x over previous
import functools
import jax
import jax.numpy as jnp
from jax import lax
from jax.experimental import pallas as pl
from jax.experimental.pallas import tpu as pltpu
from jax.experimental.pallas import tpu_sc as plsc

B, L, D = 1024, 200, 64
NSEG = 3
NW = 32
ROWS = B * L
ROWS_PER_W = ROWS // NW
CHUNK = 400
NCHUNK = ROWS_PER_W // CHUNK
PIECE = 80
NPIECE = CHUNK // PIECE
LANES = 16


def _sc_body(xf, segf, table, seg_table, pos, out,
             idxbuf, segbuf, rowbuf, posbuf, segtbuf, addbuf, sem):
    wid = lax.axis_index("s") * 2 + lax.axis_index("c")

    pltpu.sync_copy(pos, posbuf)
    pltpu.sync_copy(seg_table, segtbuf)
    segrows = [[segtbuf[s, pl.ds(LANES * k, LANES)] for k in range(D // LANES)]
               for s in range(NSEG)]

    def addloop(l, carry):
        for k in range(D // LANES):
            pk = posbuf[l, pl.ds(LANES * k, LANES)]
            for s in range(NSEG):
                addbuf[pl.ds(s * (L * D) + l * D + LANES * k, LANES)] = (
                    pk + segrows[s][k])
        return carry

    lax.fori_loop(0, L, addloop, 0)

    iota = lax.iota(jnp.int32, LANES)
    iotas = [iota + LANES * k for k in range(D // LANES)]

    def chunk_body(ci, carry):
        base = wid * ROWS_PER_W + ci * CHUNK
        pltpu.sync_copy(xf.at[pl.ds(base, CHUNK)], idxbuf)
        pltpu.sync_copy(segf.at[pl.ds(base, CHUNK)], segbuf)
        cps = [pltpu.async_copy(table.at[idxbuf.at[pl.ds(p * PIECE, PIECE)]],
                                rowbuf.at[pl.ds(p * PIECE, PIECE)], sem)
               for p in range(NPIECE)]
        for c in cps:
            c.wait()

        def grp_body(g, carry2):
            segv = segbuf[pl.ds(g * LANES, LANES)]
            lposv = lax.rem(g * LANES + iota, L)
            fbv = segv * (L * D) + lposv * D
            for j in range(LANES):
                fb = jnp.full((LANES,), fbv[j], dtype=jnp.int32)
                r = g * LANES + j
                for k in range(D // LANES):
                    iv = fb + iotas[k]
                    gv = plsc.load_gather(addbuf, [iv])
                    cur = rowbuf[r, pl.ds(LANES * k, LANES)]
                    rowbuf[r, pl.ds(LANES * k, LANES)] = cur + gv
            return carry2

        lax.fori_loop(0, CHUNK // LANES, grp_body, 0)
        pltpu.sync_copy(rowbuf, out.at[pl.ds(base, CHUNK)])
        return carry

    lax.fori_loop(0, NCHUNK, chunk_body, 0)


_sc_kernel = functools.partial(
    pl.kernel,
    mesh=plsc.VectorSubcoreMesh(core_axis_name="c", subcore_axis_name="s"),
    out_type=jax.ShapeDtypeStruct((ROWS, D), jnp.float32),
    scratch_types=[
        pltpu.VMEM((CHUNK,), jnp.int32),
        pltpu.VMEM((CHUNK,), jnp.int32),
        pltpu.VMEM((CHUNK, D), jnp.float32),
        pltpu.VMEM((L, D), jnp.float32),
        pltpu.VMEM((NSEG, D), jnp.float32),
        pltpu.VMEM((NSEG * L * D,), jnp.float32),
        pltpu.SemaphoreType.DMA,
    ],
    compiler_params=pltpu.CompilerParams(
        needs_layout_passes=False, use_tc_tiling_on_sc=False),
)(_sc_body)


def kernel(x, segment_info, table, seg_table, pos):
    xf = x.astype(jnp.int32).reshape(ROWS)
    segf = segment_info.astype(jnp.int32).reshape(ROWS)
    pos_l = pos[:L]
    out = _sc_kernel(xf, segf, table, seg_table, pos_l)
    return out.reshape(B, L, D)
